# Initial kernel scaffold; baseline (speedup 1.0000x reference)
import jax
import jax.numpy as jnp
from jax.experimental import pallas as pl


def kernel(prompt_emd, prompt_mask, cond_emd_goal, cond_mask_goal, prompt_idx_goal, cond_emd_route, cond_mask_route, prompt_idx_route, position, heading, Wq, Wk, Wv, Wkr, Wvr, Wo, W1, b1, W2, b2):
    raise NotImplementedError("write your pallas kernel here")


# speedup vs baseline: 13.9506x; 13.9506x over previous
import functools

import jax
import jax.numpy as jnp
import numpy as np
from jax.experimental import pallas as pl
from jax.experimental.pallas import tpu as pltpu

B, N, D, C, H, Hd, L, FF, M = 4, 128, 128, 64, 8, 16, 2, 256, 2
E = 3 * C
NPF = D // 4
HALF = NPF // 2

_TWO_PI = 2.0 * np.pi


def _lane_consts():
    f32 = jnp.float32
    li = jax.lax.broadcasted_iota(jnp.int32, (1, D), 1)
    off = jax.lax.rem(li, NPF)
    sin_mask = off < HALF
    gl = jax.lax.broadcasted_iota(jnp.int32, (4, D), 1)
    gr = jax.lax.broadcasted_iota(jnp.int32, (4, D), 0)
    g4 = (jax.lax.div(gl, NPF) == gr).astype(f32)
    hl = jax.lax.broadcasted_iota(jnp.int32, (H, D), 1)
    hr = jax.lax.broadcasted_iota(jnp.int32, (H, D), 0)
    hind = (jax.lax.div(hl, Hd) == hr).astype(f32)
    return sin_mask, g4, hind


def _dot(a, b):
    return jax.lax.dot_general(
        a, b, (((1,), (0,)), ((), ())),
        precision=jax.lax.Precision.HIGHEST,
        preferred_element_type=jnp.float32)


def _dot_dn(a, b, dn):
    return jax.lax.dot_general(
        a, b, (dn, ((), ())),
        precision=jax.lax.Precision.HIGHEST,
        preferred_element_type=jnp.float32)


def _bf(x):
    return x.astype(jnp.bfloat16).astype(jnp.float32)


def _dot_ref(a, b):
    return jax.lax.dot_general(
        a.astype(jnp.bfloat16), b.astype(jnp.bfloat16),
        (((1,), (0,)), ((), ())),
        preferred_element_type=jnp.float32)


def _wrap_angle(a):
    return (a + jnp.pi) % (2.0 * jnp.pi) - jnp.pi


def _body(x0_ref, maskf_ref, attr_ref, ii_ref, jj_ref, keyc_ref, keyr_ref,
          ph_ref, dimt_ref, wq_ref, wk_ref, wv_ref, wkr_ref, wvr_ref, wo_ref,
          w1_ref, b1_ref, w2_ref, b2_ref, out_ref):
    f32 = jnp.float32
    x0 = x0_ref[0]
    attr = attr_ref[0]
    ii = ii_ref[0]
    jj = jj_ref[0]
    keyc = keyc_ref[0]
    keyr = keyr_ref[0]
    ph = ph_ref[0]

    sin_mask, g4, hind = _lane_consts()

    eq = keyc == keyr
    we = jax.lax.broadcasted_iota(jnp.int32, (E, E), 0)
    wep = jax.lax.broadcasted_iota(jnp.int32, (E, E), 1)
    sameslot = (we < C) == (wep < C)
    canon_c = jnp.logical_not(jnp.any(eq & (wep > we) & sameslot,
                                      axis=1, keepdims=True))
    canon_r = jnp.logical_not(jnp.any(eq & (we > wep) & sameslot,
                                      axis=0, keepdims=True))
    has0 = jnp.any(eq & (wep < C), axis=1, keepdims=True)
    has1 = jnp.any(eq & (wep >= C), axis=1, keepdims=True)
    cnt = has0.astype(f32) + has1.astype(f32)
    pool_mat = eq.astype(f32) * canon_r.astype(f32)
    pooled = _dot(pool_mat, attr) / cnt
    repw = canon_c.astype(f32) / cnt

    lane_n = jax.lax.broadcasted_iota(jnp.int32, (E, N), 1).astype(f32)
    ohi_b = ii == lane_n
    ohi = ohi_b.astype(f32)
    ohj = (jj == lane_n).astype(f32)

    phi = _dot(ohi, ph)
    phj = _dot(ohj, ph)
    dx = phj[:, 0:1] - phi[:, 0:1]
    dy = phj[:, 1:2] - phi[:, 1:2]
    hi = phi[:, 2:3]
    hj = phj[:, 2:3]
    dist = jnp.sqrt(dx * dx + dy * dy)
    rel_ori = _wrap_angle(hj - hi)
    ci = jnp.cos(hi)
    si = jnp.sin(hi)
    cross = ci * dy - si * dx
    dotp = ci * dx + si * dy
    rov = jnp.where((dx == 0.0) & (dy == 0.0), 0.0, jnp.arctan2(cross, dotp))
    feats4 = jnp.concatenate([dist, rel_ori, rov, rov], axis=1)
    fsel = _dot(feats4, g4)
    ang = fsel / dimt_ref[...]
    pe = jnp.where(sin_mask, jnp.sin(ang), jnp.cos(ang))
    r = pooled + pe

    scale = f32(1.0 / np.sqrt(Hd))
    x = x0
    for l in range(L):
        q = _dot(x, wq_ref[l])
        xk = _dot(x, wk_ref[l])
        xv = _dot(x, wv_ref[l])
        kr = _dot_ref(r, wkr_ref[l])
        vr = _dot_ref(r, wvr_ref[l])
        qe = _dot(ohi, q)
        ke = _dot(ohj, xk) + kr
        ve = _dot(ohj, xv) + vr
        s = _dot_dn(_bf(qe) * _bf(ke), hind, (((1,), (1,)))) * scale
        t3 = jnp.where(ohi_b[:, None, :], s[:, :, None], f32(-1e9))
        m = jnp.max(t3, axis=0)
        m_at_e = _dot_dn(ohi, m, ((1,), (1,)))
        z = repw * jnp.exp(s - m_at_e)
        denom = _dot_dn(z, ohi, ((0,), (0,)))
        den_at_e = _dot_dn(ohi, denom, ((1,), (1,)))
        w = z / den_at_e
        wfull = _dot(w, hind)
        agg = _dot_dn(ohi, _bf(wfull) * _bf(ve), ((0,), (0,)))
        x = x + _dot_ref(agg, wo_ref[l])
        mu = jnp.mean(x, axis=1, keepdims=True)
        var = jnp.mean((x - mu) ** 2, axis=1, keepdims=True)
        hn = (x - mu) / jnp.sqrt(var + 1e-5)
        ff = _dot(jnp.maximum(_dot(hn, w1_ref[l]) + b1_ref[l], 0.0),
                  w2_ref[l])
        x = x + ff + b2_ref[l]

    mk = maskf_ref[0]
    out_ref[0] = x * mk + x0 * (1.0 - mk)


@functools.partial(jax.jit, static_argnames=("interpret",))
def _run(x0, maskf, attr, iif, jjf, keyc, keyr, ph, dimt,
         Wq, Wk, Wv, Wkr, Wvr, Wo, W1, b1r, W2, b2r, interpret=False):
    bspec = lambda shape: pl.BlockSpec(
        (1,) + shape, lambda b: (b,) + (0,) * len(shape))
    wspec = lambda shape: pl.BlockSpec(shape, lambda b: (0,) * len(shape))
    return pl.pallas_call(
        _body,
        grid=(B,),
        in_specs=[
            bspec((N, D)),
            bspec((N, 1)),
            bspec((E, D)),
            bspec((E, 1)),
            bspec((E, 1)),
            bspec((E, 1)),
            bspec((1, E)),
            bspec((N, 8)),
            wspec((1, D)),
            wspec((L, D, H * Hd)),
            wspec((L, D, H * Hd)),
            wspec((L, D, H * Hd)),
            wspec((L, D, H * Hd)),
            wspec((L, D, H * Hd)),
            wspec((L, H * Hd, D)),
            wspec((L, D, FF)),
            wspec((L, 1, FF)),
            wspec((L, FF, D)),
            wspec((L, 1, D)),
        ],
        out_specs=bspec((N, D)),
        out_shape=jax.ShapeDtypeStruct((B, N, D), jnp.float32),
        interpret=interpret,
    )(x0, maskf, attr, iif, jjf, keyc, keyr, ph, dimt,
      Wq, Wk, Wv, Wkr, Wvr, Wo, W1, b1r, W2, b2r)


def kernel(prompt_emd, prompt_mask, cond_emd_goal, cond_mask_goal,
           prompt_idx_goal, cond_emd_route, cond_mask_route, prompt_idx_route,
           position, heading, Wq, Wk, Wv, Wkr, Wvr, Wo, W1, b1, W2, b2):
    f32 = jnp.float32
    sg = prompt_idx_goal[..., 0]
    sr = prompt_idx_route[..., 0]
    tr = prompt_idx_route[..., 1]
    ii = jnp.concatenate([sg, sr, tr], axis=1)
    jj = jnp.concatenate([sg, tr, sr], axis=1)
    key = (ii * N + jj).astype(f32)
    iif = ii.astype(f32)[..., None]
    jjf = jj.astype(f32)[..., None]
    keyc = key[..., None]
    keyr = key[:, None, :]
    attr = jnp.concatenate(
        [cond_emd_goal, cond_emd_route[..., :D], cond_emd_route[..., D:]],
        axis=1)
    ph = jnp.concatenate(
        [position, heading, jnp.zeros((B, N, 5), f32)], axis=-1)
    maskf = prompt_mask.astype(f32)[..., None]
    dim_t = 10000.0 ** (2.0 * jnp.arange(HALF) / NPF)
    dimt = jnp.tile(jnp.concatenate([dim_t, dim_t]), 4)[None, :].astype(f32)
    b1r = b1[:, None, :]
    b2r = b2[:, None, :]
    return _run(prompt_emd, maskf, attr, iif, jjf, keyc, keyr, ph, dimt,
                Wq, Wk, Wv, Wkr, Wvr, Wo, W1, b1r, W2, b2r)
